# Initial kernel scaffold; baseline (speedup 1.0000x reference)
import jax
import jax.numpy as jnp
from jax.experimental import pallas as pl


def kernel(loss, gt_boxes2d):
    raise NotImplementedError("write your pallas kernel here")


# speedup vs baseline: 1.0864x; 1.0864x over previous
import functools

import jax
import jax.numpy as jnp
from jax import lax
from jax.experimental import pallas as pl
from jax.experimental.pallas import tpu as pltpu
from jax.experimental.pallas import tpu_sc as plsc

B, H, W = 16, 512, 512
NBOX = 20
NBOXP = 32
NWORK = 32
ROWS_PER_W = H // 2
R_CHUNK = 64
N_CHUNKS = ROWS_PER_W // R_CHUNK
WCH = W // 16
NPIX = B * H * W
FG_EXTRA = 12.0


def _body(loss_hbm, bx_hbm, out_hbm, bx_v, buf_v, cov_v, acc_v):
    c = lax.axis_index("c")
    s = lax.axis_index("s")
    wid = s * 2 + c
    batch = wid // 2
    row0 = (wid % 2) * ROWS_PER_W

    pltpu.sync_copy(bx_hbm.at[pl.ds(batch * 128, 128)], bx_v)

    iota = lax.iota(jnp.int32, 16)
    v1a = bx_v[pl.ds(0, 16)]
    v1b = bx_v[pl.ds(16, 16)]
    v2a = bx_v[pl.ds(32, 16)]
    v2b = bx_v[pl.ds(48, 16)]
    u1a = bx_v[pl.ds(64, 16)]
    u1b = bx_v[pl.ds(80, 16)]
    u2a = bx_v[pl.ds(96, 16)]
    u2b = bx_v[pl.ds(112, 16)]

    zero16 = jnp.zeros((16,), jnp.float32)
    acc_v[pl.ds(0, 16)] = zero16
    acc_v[pl.ds(16, 16)] = zero16

    def chunk_body(k, _):
        base = (batch * H + row0 + k * R_CHUNK) * W
        pltpu.sync_copy(loss_hbm.at[pl.ds(base, R_CHUNK * W)], buf_v)

        def row_body(r, _):
            v = row0 + k * R_CHUNK + r
            vv = jnp.full((16,), v, jnp.int32)
            acta = (vv >= v1a) & (vv < v2a)
            actb = (vv >= v1b) & (vv < v2b)
            have = jnp.any(acta) | jnp.any(actb)
            event = (v == row0) | jnp.any(
                (vv == v1a) | (vv == v2a) | (vv == v1b) | (vv == v2b))
            off = r * W

            vals = [buf_v[pl.ds(off + i * 16, 16)] for i in range(WCH)]
            while len(vals) > 1:
                nxt = [vals[i] + vals[i + 1] for i in range(0, len(vals) - 1, 2)]
                if len(vals) % 2:
                    nxt.append(vals[-1])
                vals = nxt
            acc_v[pl.ds(0, 16)] = acc_v[pl.ds(0, 16)] + vals[0]

            @pl.when(have)
            def _fg_row():
                @pl.when(event)
                def _recompute_cov():
                    eu1a = jnp.where(acta, u1a, 0)
                    eu1b = jnp.where(actb, u1b, 0)
                    eu2a = jnp.where(acta, u2a, 0)
                    eu2b = jnp.where(actb, u2b, 0)
                    bounds = [(eu1a[j], eu2a[j]) for j in range(16)]
                    bounds += [(eu1b[j], eu2b[j]) for j in range(NBOXP - 16)]

                    def col_body(ci, _):
                        cols = iota + ci * 16
                        cov = jnp.zeros((16,), jnp.bool_)
                        for e1, e2 in bounds:
                            cov = cov | ((cols >= e1) & (cols < e2))
                        cov_v[pl.ds(ci * 16, 16)] = jnp.where(
                            cov, jnp.float32(FG_EXTRA), jnp.float32(0.0))
                        return 0

                    lax.fori_loop(0, WCH, col_body, 0)

                fvals = [buf_v[pl.ds(off + i * 16, 16)] * cov_v[pl.ds(i * 16, 16)]
                         for i in range(WCH)]
                while len(fvals) > 1:
                    nxt = [fvals[i] + fvals[i + 1] for i in range(0, len(fvals) - 1, 2)]
                    if len(fvals) % 2:
                        nxt.append(fvals[-1])
                    fvals = nxt
                acc_v[pl.ds(16, 16)] = acc_v[pl.ds(16, 16)] + fvals[0]

            return 0

        lax.fori_loop(0, R_CHUNK, row_body, 0)
        return 0

    lax.fori_loop(0, N_CHUNKS, chunk_body, 0)

    acc_v[pl.ds(0, 16)] = acc_v[pl.ds(0, 16)] + acc_v[pl.ds(16, 16)]
    pltpu.sync_copy(acc_v.at[pl.ds(0, 16)], out_hbm.at[pl.ds(wid * 16, 16)])


@functools.cache
def _build_sc_kernel():
    mesh = plsc.VectorSubcoreMesh(core_axis_name="c", subcore_axis_name="s")
    return pl.kernel(
        _body,
        out_type=jax.ShapeDtypeStruct((NWORK * 16,), jnp.float32),
        mesh=mesh,
        compiler_params=pltpu.CompilerParams(needs_layout_passes=False),
        scratch_types=[
            pltpu.VMEM((128,), jnp.int32),
            pltpu.VMEM((R_CHUNK * W,), jnp.float32),
            pltpu.VMEM((W,), jnp.float32),
            pltpu.VMEM((32,), jnp.float32),
        ],
    )


def kernel(loss, gt_boxes2d):
    xy1 = jnp.floor(gt_boxes2d[:, :, :2])
    xy2 = jnp.ceil(gt_boxes2d[:, :, 2:])
    gbi = jnp.concatenate([xy1, xy2], axis=-1).astype(jnp.int32)
    pad = jnp.zeros((B, NBOXP - NBOX), jnp.int32)
    v1 = jnp.concatenate([gbi[:, :, 1], pad], axis=1)
    v2 = jnp.concatenate([gbi[:, :, 3], pad], axis=1)
    u1 = jnp.concatenate([gbi[:, :, 0], pad], axis=1)
    u2 = jnp.concatenate([gbi[:, :, 2], pad], axis=1)
    bx = jnp.concatenate([v1, v2, u1, u2], axis=1).reshape(-1)
    partials = _build_sc_kernel()(loss.reshape(-1), bx)
    return partials.sum() / jnp.float32(NPIX)


# speedup vs baseline: 1.6181x; 1.4893x over previous
import functools

import jax
import jax.numpy as jnp
from jax import lax
from jax.experimental import pallas as pl
from jax.experimental.pallas import tpu as pltpu
from jax.experimental.pallas import tpu_sc as plsc

B, H, W = 16, 512, 512
NBOX = 20
NBOXP = 32
NWORK = 32
ROWS_PER_W = H // 2
R_CHUNK = 32
N_CHUNKS = ROWS_PER_W // R_CHUNK
WCH = W // 16
NPIX = B * H * W
FG_EXTRA = 12.0


def _row_weighted(buf_v, rr, cov_v):
    vals = [buf_v[rr, pl.ds(i * 16, 16)] * cov_v[pl.ds(i * 16, 16)]
            for i in range(WCH)]
    while len(vals) > 1:
        nxt = [vals[i] + vals[i + 1] for i in range(0, len(vals) - 1, 2)]
        if len(vals) % 2:
            nxt.append(vals[-1])
        vals = nxt
    return vals[0]


def _body(loss_hbm, bx_hbm, out_hbm, bx_v, buf_v, cov_v, acc_v, sem0, sem1):
    c = lax.axis_index("c")
    s = lax.axis_index("s")
    wid = s * 2 + c
    batch = wid // 2
    row0 = (wid % 2) * ROWS_PER_W
    sems = (sem0, sem1)

    pltpu.sync_copy(bx_hbm.at[pl.ds(batch * 128, 128)], bx_v)

    iota = lax.iota(jnp.int32, 16)
    v1a = bx_v[pl.ds(0, 16)]
    v1b = bx_v[pl.ds(16, 16)]
    v2a = bx_v[pl.ds(32, 16)]
    v2b = bx_v[pl.ds(48, 16)]
    u1a = bx_v[pl.ds(64, 16)]
    u1b = bx_v[pl.ds(80, 16)]
    u2a = bx_v[pl.ds(96, 16)]
    u2b = bx_v[pl.ds(112, 16)]

    zero16 = jnp.zeros((16,), jnp.float32)
    acc_v[pl.ds(0, 16)] = zero16
    acc_v[pl.ds(16, 16)] = zero16

    def _dma(k, slot):
        return pltpu.make_async_copy(
            loss_hbm.at[batch, pl.ds(row0 + k * R_CHUNK, R_CHUNK), :],
            buf_v.at[slot], sems[slot])

    _dma(0, 0).start()

    def _compute_chunk(k, slot):
        def row_sum(r, accs):
            a0, a1, a2, a3 = accs
            for i in range(0, WCH, 4):
                a0 = a0 + buf_v[slot, r, pl.ds(i * 16, 16)]
                a1 = a1 + buf_v[slot, r, pl.ds((i + 1) * 16, 16)]
                a2 = a2 + buf_v[slot, r, pl.ds((i + 2) * 16, 16)]
                a3 = a3 + buf_v[slot, r, pl.ds((i + 3) * 16, 16)]
            return a0, a1, a2, a3

        a0, a1, a2, a3 = lax.fori_loop(
            0, R_CHUNK, row_sum, (zero16, zero16, zero16, zero16))
        acc_v[pl.ds(0, 16)] = acc_v[pl.ds(0, 16)] + ((a0 + a1) + (a2 + a3))

        for g in range(R_CHUNK // 16):
            gmin = row0 + k * R_CHUNK + g * 16
            gmax16 = jnp.full((16,), gmin + 15, jnp.int32)
            gmin16 = jnp.full((16,), gmin, jnp.int32)
            ghit = jnp.any((v1a <= gmax16) & (v2a > gmin16)) | jnp.any(
                (v1b <= gmax16) & (v2b > gmin16))

            @pl.when(ghit)
            def _group():
                def row_fg(r, _):
                    v = gmin + r
                    vv = jnp.full((16,), v, jnp.int32)
                    acta = (vv >= v1a) & (vv < v2a)
                    actb = (vv >= v1b) & (vv < v2b)
                    have = jnp.any(acta) | jnp.any(actb)

                    @pl.when(have)
                    def _fg_row():
                        event = (v == row0) | jnp.any(
                            (vv == v1a) | (vv == v2a)
                            | (vv == v1b) | (vv == v2b))

                        @pl.when(event)
                        def _recompute_cov():
                            eu1a = jnp.where(acta, u1a, 0)
                            eu1b = jnp.where(actb, u1b, 0)
                            eu2a = jnp.where(acta, u2a, 0)
                            eu2b = jnp.where(actb, u2b, 0)
                            bounds = [(eu1a[j], eu2a[j]) for j in range(16)]
                            bounds += [(eu1b[j], eu2b[j])
                                       for j in range(NBOXP - 16)]

                            def col_body(ci, _):
                                cols = iota + ci * 16
                                cov = jnp.zeros((16,), jnp.bool_)
                                for e1, e2 in bounds:
                                    cov = cov | ((cols >= e1) & (cols < e2))
                                cov_v[pl.ds(ci * 16, 16)] = jnp.where(
                                    cov, jnp.float32(FG_EXTRA),
                                    jnp.float32(0.0))
                                return 0

                            lax.fori_loop(0, WCH, col_body, 0)

                        rr = g * 16 + r
                        acc_v[pl.ds(16, 16)] = (
                            acc_v[pl.ds(16, 16)]
                            + _row_weighted(buf_v.at[slot], rr, cov_v))

                    return 0

                lax.fori_loop(0, 16, row_fg, 0)

    def ring(i, _):
        k2 = i * 2
        for b in (0, 1):
            k = k2 + b
            _dma(k, b).wait()

            @pl.when(k + 1 < N_CHUNKS)
            def _start_next():
                _dma(k + 1, 1 - b).start()

            _compute_chunk(k, b)
        return 0

    lax.fori_loop(0, N_CHUNKS // 2, ring, 0)

    acc_v[pl.ds(0, 16)] = acc_v[pl.ds(0, 16)] + acc_v[pl.ds(16, 16)]
    pltpu.sync_copy(acc_v.at[pl.ds(0, 16)], out_hbm.at[pl.ds(wid * 16, 16)])


@functools.cache
def _build_sc_kernel():
    mesh = plsc.VectorSubcoreMesh(core_axis_name="c", subcore_axis_name="s")
    return pl.kernel(
        _body,
        out_type=jax.ShapeDtypeStruct((NWORK * 16,), jnp.float32),
        mesh=mesh,
        compiler_params=pltpu.CompilerParams(
            needs_layout_passes=False, use_tc_tiling_on_sc=True),
        scratch_types=[
            pltpu.VMEM((128,), jnp.int32),
            pltpu.VMEM((2, R_CHUNK, W), jnp.float32),
            pltpu.VMEM((W,), jnp.float32),
            pltpu.VMEM((32,), jnp.float32),
            pltpu.SemaphoreType.DMA,
            pltpu.SemaphoreType.DMA,
        ],
    )


def kernel(loss, gt_boxes2d):
    xy1 = jnp.floor(gt_boxes2d[:, :, :2])
    xy2 = jnp.ceil(gt_boxes2d[:, :, 2:])
    gbi = jnp.concatenate([xy1, xy2], axis=-1).astype(jnp.int32)
    pad = jnp.zeros((B, NBOXP - NBOX), jnp.int32)
    v1 = jnp.concatenate([gbi[:, :, 1], pad], axis=1)
    v2 = jnp.concatenate([gbi[:, :, 3], pad], axis=1)
    u1 = jnp.concatenate([gbi[:, :, 0], pad], axis=1)
    u2 = jnp.concatenate([gbi[:, :, 2], pad], axis=1)
    bx = jnp.concatenate([v1, v2, u1, u2], axis=1).reshape(-1)
    partials = _build_sc_kernel()(loss, bx)
    return partials.sum() / jnp.float32(NPIX)


# speedup vs baseline: 1.6943x; 1.0471x over previous
import functools

import jax
import jax.numpy as jnp
from jax import lax
from jax.experimental import pallas as pl
from jax.experimental.pallas import tpu as pltpu
from jax.experimental.pallas import tpu_sc as plsc

B, H, W = 16, 512, 512
NBOX = 20
NBOXP = 32
NWORK = 32
ROWS_PER_W = H // 2
R_CHUNK = 64
N_CHUNKS = ROWS_PER_W // R_CHUNK
WCH = W // 16
NPIX = B * H * W
FG_EXTRA = 12.0


def _row_weighted(buf_v, rr, cov_v):
    vals = [buf_v[rr, pl.ds(i * 16, 16)] * cov_v[pl.ds(i * 16, 16)]
            for i in range(WCH)]
    while len(vals) > 1:
        nxt = [vals[i] + vals[i + 1] for i in range(0, len(vals) - 1, 2)]
        if len(vals) % 2:
            nxt.append(vals[-1])
        vals = nxt
    return vals[0]


def _floor_i(x):
    t = x.astype(jnp.int32)
    return t - jnp.where(t.astype(jnp.float32) > x, 1, 0)


def _ceil_i(x):
    t = x.astype(jnp.int32)
    return t + jnp.where(x > t.astype(jnp.float32), 1, 0)


def _body(loss_hbm, gt_hbm, out_hbm, bxr_v, buf_v, cov_v, acc_v, sem0, sem1):
    c = lax.axis_index("c")
    s = lax.axis_index("s")
    wid = s * 2 + c
    batch = wid // 2
    row0 = (wid % 2) * ROWS_PER_W
    sems = (sem0, sem1)

    pltpu.sync_copy(gt_hbm.at[batch], bxr_v)

    iota = lax.iota(jnp.int32, 16)
    zero_i = jnp.zeros((16,), jnp.int32)
    idxb = jnp.where(iota < NBOX - 16, iota + 16, 0)
    validb = iota < NBOX - 16

    def _col(col, idx):
        return plsc.load_gather(bxr_v, [idx, jnp.full((16,), col, jnp.int32)])

    u1a = _floor_i(_col(0, iota))
    v1a = _floor_i(_col(1, iota))
    u2a = _ceil_i(_col(2, iota))
    v2a = _ceil_i(_col(3, iota))
    u1b = jnp.where(validb, _floor_i(_col(0, idxb)), zero_i)
    v1b = jnp.where(validb, _floor_i(_col(1, idxb)), zero_i)
    u2b = jnp.where(validb, _ceil_i(_col(2, idxb)), zero_i)
    v2b = jnp.where(validb, _ceil_i(_col(3, idxb)), zero_i)

    zero16 = jnp.zeros((16,), jnp.float32)
    acc_v[pl.ds(0, 16)] = zero16
    acc_v[pl.ds(16, 16)] = zero16

    def _dma(k, slot):
        return pltpu.make_async_copy(
            loss_hbm.at[batch, pl.ds(row0 + k * R_CHUNK, R_CHUNK), :],
            buf_v.at[slot], sems[slot])

    _dma(0, 0).start()

    def _compute_chunk(k, slot):
        def row_sum(r, accs):
            a0, a1, a2, a3 = accs
            for i in range(0, WCH, 4):
                a0 = a0 + buf_v[slot, r, pl.ds(i * 16, 16)]
                a1 = a1 + buf_v[slot, r, pl.ds((i + 1) * 16, 16)]
                a2 = a2 + buf_v[slot, r, pl.ds((i + 2) * 16, 16)]
                a3 = a3 + buf_v[slot, r, pl.ds((i + 3) * 16, 16)]
            return a0, a1, a2, a3

        a0, a1, a2, a3 = lax.fori_loop(
            0, R_CHUNK, row_sum, (zero16, zero16, zero16, zero16), unroll=2)
        acc_v[pl.ds(0, 16)] = acc_v[pl.ds(0, 16)] + ((a0 + a1) + (a2 + a3))

        for g in range(R_CHUNK // 16):
            gmin = row0 + k * R_CHUNK + g * 16
            gmax16 = jnp.full((16,), gmin + 15, jnp.int32)
            gmin16 = jnp.full((16,), gmin, jnp.int32)
            ghit = jnp.any((v1a <= gmax16) & (v2a > gmin16)) | jnp.any(
                (v1b <= gmax16) & (v2b > gmin16))

            @pl.when(ghit)
            def _group():
                def row_fg(r, _):
                    v = gmin + r
                    vv = jnp.full((16,), v, jnp.int32)
                    acta = (vv >= v1a) & (vv < v2a)
                    actb = (vv >= v1b) & (vv < v2b)
                    have = jnp.any(acta) | jnp.any(actb)

                    @pl.when(have)
                    def _fg_row():
                        event = (v == row0) | jnp.any(
                            (vv == v1a) | (vv == v2a)
                            | (vv == v1b) | (vv == v2b))

                        @pl.when(event)
                        def _recompute_cov():
                            eu1a = jnp.where(acta, u1a, 0)
                            eu1b = jnp.where(actb, u1b, 0)
                            eu2a = jnp.where(acta, u2a, 0)
                            eu2b = jnp.where(actb, u2b, 0)
                            bounds = [(eu1a[j], eu2a[j]) for j in range(16)]
                            bounds += [(eu1b[j], eu2b[j])
                                       for j in range(NBOXP - 16)]

                            def col_body(ci, _):
                                cols = iota + ci * 16
                                cov = jnp.zeros((16,), jnp.bool_)
                                for e1, e2 in bounds:
                                    cov = cov | ((cols >= e1) & (cols < e2))
                                cov_v[pl.ds(ci * 16, 16)] = jnp.where(
                                    cov, jnp.float32(FG_EXTRA),
                                    jnp.float32(0.0))
                                return 0

                            lax.fori_loop(0, WCH, col_body, 0)

                        rr = g * 16 + r
                        acc_v[pl.ds(16, 16)] = (
                            acc_v[pl.ds(16, 16)]
                            + _row_weighted(buf_v.at[slot], rr, cov_v))

                    return 0

                lax.fori_loop(0, 16, row_fg, 0)

    def ring(i, _):
        k2 = i * 2
        for b in (0, 1):
            k = k2 + b
            _dma(k, b).wait()

            @pl.when(k + 1 < N_CHUNKS)
            def _start_next():
                _dma(k + 1, 1 - b).start()

            _compute_chunk(k, b)
        return 0

    lax.fori_loop(0, N_CHUNKS // 2, ring, 0)

    acc_v[pl.ds(0, 16)] = acc_v[pl.ds(0, 16)] + acc_v[pl.ds(16, 16)]
    pltpu.sync_copy(acc_v.at[pl.ds(0, 16)], out_hbm.at[pl.ds(wid * 16, 16)])


@functools.cache
def _build_sc_kernel():
    mesh = plsc.VectorSubcoreMesh(core_axis_name="c", subcore_axis_name="s")
    return pl.kernel(
        _body,
        out_type=jax.ShapeDtypeStruct((NWORK * 16,), jnp.float32),
        mesh=mesh,
        compiler_params=pltpu.CompilerParams(
            needs_layout_passes=False, use_tc_tiling_on_sc=True),
        scratch_types=[
            pltpu.VMEM((NBOX, 4), jnp.float32),
            pltpu.VMEM((2, R_CHUNK, W), jnp.float32),
            pltpu.VMEM((W,), jnp.float32),
            pltpu.VMEM((32,), jnp.float32),
            pltpu.SemaphoreType.DMA,
            pltpu.SemaphoreType.DMA,
        ],
    )


def kernel(loss, gt_boxes2d):
    partials = _build_sc_kernel()(loss, gt_boxes2d)
    return partials.sum() / jnp.float32(NPIX)
